# baseline (speedup 1.0000x reference)
import functools

import jax
import jax.numpy as jnp
from jax import lax
from jax.experimental import pallas as pl
from jax.experimental.pallas import tpu as pltpu
from jax.experimental.pallas import tpu_sc as plsc

NC = 2
NS = 16
L = 16
NW = NC * NS

B = 16384
D = 64
BPW = B // NW
CHUNK = 128
NCHUNK = BPW // CHUNK


def _sc_body(uidx_hbm, iidx_hbm, ut_hbm, it_hbm,
             out_hbm, ue_hbm, ie_hbm,
             uix, iix, urows, irows, dots, su, si, so):
    wid = lax.axis_index("s") * NC + lax.axis_index("c")
    base = wid * BPW

    pltpu.sync_copy(uidx_hbm.at[pl.ds(base, BPW)], uix)
    pltpu.sync_copy(iidx_hbm.at[pl.ds(base, BPW)], iix)

    copies = []
    for k in range(NCHUNK):
        sl = pl.ds(k * CHUNK, CHUNK)
        copies.append(pltpu.async_copy(ut_hbm.at[uix.at[sl]], urows.at[sl], su))
        copies.append(pltpu.async_copy(it_hbm.at[iix.at[sl]], irows.at[sl], si))
    for c in copies:
        c.wait()

    emb_copies = [
        pltpu.async_copy(urows, ue_hbm.at[pl.ds(base, BPW)], so),
        pltpu.async_copy(irows, ie_hbm.at[pl.ds(base, BPW)], so),
    ]

    lane = lax.iota(jnp.int32, L)

    def group(g, _):
        rows = g * L + lane
        acc = jnp.zeros((L,), jnp.float32)
        for j in range(D):
            col = jnp.full((L,), j, jnp.int32)
            u = plsc.load_gather(urows, [rows, col])
            v = plsc.load_gather(irows, [rows, col])
            acc = acc + u * v
        dots[pl.ds(g * L, L)] = acc
        return 0

    lax.fori_loop(0, BPW // L, group, 0)

    pltpu.sync_copy(dots, out_hbm.at[pl.ds(base, BPW)])
    for c in emb_copies:
        c.wait()


def kernel(x, user_table, item_table):
    uidx = x[:, 0]
    iidx = x[:, 1]

    mesh = plsc.VectorSubcoreMesh(
        core_axis_name="c", subcore_axis_name="s",
        num_cores=NC, num_subcores=NS)

    run = functools.partial(
        pl.kernel,
        out_type=(
            jax.ShapeDtypeStruct((B,), jnp.float32),
            jax.ShapeDtypeStruct((B, D), jnp.float32),
            jax.ShapeDtypeStruct((B, D), jnp.float32),
        ),
        mesh=mesh,
        compiler_params=pltpu.CompilerParams(
            needs_layout_passes=False, use_tc_tiling_on_sc=False),
        scratch_types=[
            pltpu.VMEM((BPW,), jnp.int32),
            pltpu.VMEM((BPW,), jnp.int32),
            pltpu.VMEM((BPW, D), jnp.float32),
            pltpu.VMEM((BPW, D), jnp.float32),
            pltpu.VMEM((BPW,), jnp.float32),
            pltpu.SemaphoreType.DMA,
            pltpu.SemaphoreType.DMA,
            pltpu.SemaphoreType.DMA,
        ],
    )(_sc_body)

    out_flat, ue, ie = run(uidx, iidx, user_table, item_table)
    return (out_flat[:, None], ue, ie)
